# Initial kernel scaffold; baseline (speedup 1.0000x reference)
import jax
import jax.numpy as jnp
from jax.experimental import pallas as pl


def kernel(x_os_val, y_os_val, z_os_val, i_val, sigma, b, ch, z, y, x):
    raise NotImplementedError("write your pallas kernel here")


# speedup vs baseline: 33.0216x; 33.0216x over previous
import jax
import jax.numpy as jnp
from jax import lax
from jax.experimental import pallas as pl
from jax.experimental.pallas import tpu as pltpu
from jax.experimental.pallas import tpu_sc as plsc

N_EM = 2000
PSF = 21
PAD = PSF // 2
NB, NH, NW_, ND = 4, 128, 128, 64
SCALE_ = 10000.0
NC, NS = 2, 16
NWORK = NC * NS
ZS = 8
SLAB = ZS * NW_ * ND
NSLAB = (NB * NH) // ZS
SLABS_PER_B = NH // ZS
REPS = NSLAB // NWORK
NPATCH = PSF * PSF
CHUNKS = (NPATCH + 15) // 16
EPAD = N_EM + 16


def _sload(ref, i):
    return ref[pl.ds(i, 16)][0]


def _sc_body(xos_h, yos_h, zos_h, ival_h, sig_h, b_h, z_h, y_h, x_h,
             kytab_h, kxtab_h, zero_h, out_h,
             slab, xos, yos, zos, ival, eb, ez, ey, ex,
             kytab, kxtab, gy, gx, az, pval, pidx, sig):
    wid = lax.axis_index("s") * NC + lax.axis_index("c")

    pltpu.sync_copy(xos_h, xos.at[pl.ds(0, N_EM)])
    pltpu.sync_copy(yos_h, yos.at[pl.ds(0, N_EM)])
    pltpu.sync_copy(zos_h, zos.at[pl.ds(0, N_EM)])
    pltpu.sync_copy(ival_h, ival.at[pl.ds(0, N_EM)])
    pltpu.sync_copy(b_h, eb.at[pl.ds(0, N_EM)])
    pltpu.sync_copy(z_h, ez.at[pl.ds(0, N_EM)])
    pltpu.sync_copy(y_h, ey.at[pl.ds(0, N_EM)])
    pltpu.sync_copy(x_h, ex.at[pl.ds(0, N_EM)])
    pltpu.sync_copy(kytab_h, kytab)
    pltpu.sync_copy(kxtab_h, kxtab)
    pltpu.sync_copy(sig_h, sig)

    sig_v = sig[pl.ds(0, 16)]
    inv2s2 = 0.5 / (sig_v * sig_v)
    iota = lax.iota(jnp.int32, 16)

    for rep in range(REPS):
        slab_id = wid + rep * NWORK
        sb = slab_id // SLABS_PER_B
        z0 = (slab_id % SLABS_PER_B) * ZS
        pltpu.sync_copy(zero_h, slab)

        def escan(e, carry):
            be = _sload(eb, e)
            ze = _sload(ez, e)
            hit = (be == sb) & (ze >= z0 - PAD) & (ze <= z0 + ZS - 1 + PAD)

            @pl.when(hit)
            def _():
                ye = _sload(ey, e)
                xe = _sload(ex, e)
                zos_e = _sload(zos, e)
                yos_e = _sload(yos, e)
                xos_e = _sload(xos, e)
                amp = SCALE_ * jnp.maximum(_sload(ival, e), 0.0)

                for c in range(2):
                    k = iota + c * 16
                    kf = k.astype(jnp.float32)
                    valid = k < PSF
                    dz = kf - (10.0 + zos_e)
                    gzv = jnp.exp(-(dz * dz - zos_e * zos_e) * inv2s2) * amp
                    az[pl.ds(c * 16, 16)] = jnp.where(valid, gzv, 0.0)
                    dy = kf - (10.0 + yos_e)
                    gyv = jnp.exp(-(dy * dy - yos_e * yos_e) * inv2s2)
                    gy[pl.ds(c * 16, 16)] = jnp.where(valid, gyv, 0.0)
                    dx = kf - (10.0 + xos_e)
                    gxv = jnp.exp(-(dx * dx - xos_e * xos_e) * inv2s2)
                    gx[pl.ds(c * 16, 16)] = jnp.where(valid, gxv, 0.0)

                y0 = ye - PAD
                x0 = xe - PAD
                for c in range(CHUNKS):
                    kyv = kytab[pl.ds(c * 16, 16)]
                    kxv = kxtab[pl.ds(c * 16, 16)]
                    vy = plsc.load_gather(gy, [kyv])
                    vx = plsc.load_gather(gx, [kxv])
                    yy = y0 + kyv
                    xx = x0 + kxv
                    inb = (yy >= 0) & (yy < NH) & (xx >= 0) & (xx < ND)
                    pval[pl.ds(c * 16, 16)] = jnp.where(inb, vy * vx, 0.0)
                    pidx[pl.ds(c * 16, 16)] = jnp.where(inb, yy * ND + xx, 0)

                lo = jnp.maximum(ze - PAD, z0)
                hi = jnp.minimum(ze + PAD, z0 + ZS - 1)

                def zbody(oz, cc):
                    kz = oz - (ze - PAD)
                    s = _sload(az, kz)
                    zb = (oz - z0) * (NW_ * ND)
                    for c in range(CHUNKS):
                        v = pval[pl.ds(c * 16, 16)] * s
                        ii = pidx[pl.ds(c * 16, 16)] + zb
                        plsc.addupdate_scatter(slab, [ii], v)
                    return cc

                lax.fori_loop(lo, hi + 1, zbody, 0)

            return carry

        lax.fori_loop(0, N_EM, escan, 0)
        pltpu.sync_copy(slab, out_h.at[pl.ds(slab_id * SLAB, SLAB)])


def kernel(x_os_val, y_os_val, z_os_val, i_val, sigma, b, ch, z, y, x):
    del ch
    lin = jnp.arange(CHUNKS * 16, dtype=jnp.int32)
    kytab = jnp.where(lin < NPATCH, lin // PSF, 0)
    kxtab = jnp.where(lin < NPATCH, lin % PSF, 31)
    sig16 = jnp.full((16,), sigma, dtype=jnp.float32)
    zero = jnp.zeros((SLAB,), dtype=jnp.float32)

    mesh = plsc.VectorSubcoreMesh(core_axis_name="c", subcore_axis_name="s",
                                  num_cores=NC, num_subcores=NS)
    out = pl.kernel(
        _sc_body,
        out_type=jax.ShapeDtypeStruct((NB * NH * NW_ * ND,), jnp.float32),
        mesh=mesh,
        compiler_params=pltpu.CompilerParams(needs_layout_passes=False),
        scratch_types=[
            pltpu.VMEM((SLAB,), jnp.float32),
            pltpu.VMEM((EPAD,), jnp.float32),
            pltpu.VMEM((EPAD,), jnp.float32),
            pltpu.VMEM((EPAD,), jnp.float32),
            pltpu.VMEM((EPAD,), jnp.float32),
            pltpu.VMEM((EPAD,), jnp.int32),
            pltpu.VMEM((EPAD,), jnp.int32),
            pltpu.VMEM((EPAD,), jnp.int32),
            pltpu.VMEM((EPAD,), jnp.int32),
            pltpu.VMEM((CHUNKS * 16,), jnp.int32),
            pltpu.VMEM((CHUNKS * 16,), jnp.int32),
            pltpu.VMEM((32,), jnp.float32),
            pltpu.VMEM((32,), jnp.float32),
            pltpu.VMEM((48,), jnp.float32),
            pltpu.VMEM((CHUNKS * 16,), jnp.float32),
            pltpu.VMEM((CHUNKS * 16,), jnp.int32),
            pltpu.VMEM((16,), jnp.float32),
        ],
    )(x_os_val, y_os_val, z_os_val, i_val, sig16,
      b.astype(jnp.int32), z.astype(jnp.int32), y.astype(jnp.int32),
      x.astype(jnp.int32), kytab, kxtab, zero)
    return out.reshape(NB, 1, NH, NW_, ND)


# speedup vs baseline: 84.6500x; 2.5635x over previous
import jax
import jax.numpy as jnp
from jax import lax
from jax.experimental import pallas as pl
from jax.experimental.pallas import tpu as pltpu
from jax.experimental.pallas import tpu_sc as plsc

N_EM = 2000
PSF = 21
PAD = PSF // 2
NB, NH, NW_, ND = 4, 128, 128, 64
SCALE_ = 10000.0
NC, NS = 2, 16
NWORK = NC * NS
ZS = 8
SLICE = NW_ * ND
SLAB = ZS * SLICE
NSLAB = (NB * NH) // ZS
SLABS_PER_B = NH // ZS
REPS = NSLAB // NWORK
NPATCH = PSF * PSF
CHUNKS = (NPATCH + 15) // 16
NGRP = N_EM // 16
EPAD = N_EM + 16
AZOFF = 16
AZLEN = 64


def _sload(ref, i):
    return ref[pl.ds(i, 16)][0]


def _sc_body(xos_h, yos_h, zos_h, ival_h, sig_h, b_h, z_h, y_h, x_h,
             kyi_h, kxi_h, kyf_h, kxf_h, zero_h, out_h,
             slab, xos, yos, zos, ival, eb, ez, ey, ex,
             kyi, kxi, kyf, kxf, az, pval, pidx, elist, sig):
    wid = lax.axis_index("s") * NC + lax.axis_index("c")

    pltpu.sync_copy(xos_h, xos.at[pl.ds(0, N_EM)])
    pltpu.sync_copy(yos_h, yos.at[pl.ds(0, N_EM)])
    pltpu.sync_copy(zos_h, zos.at[pl.ds(0, N_EM)])
    pltpu.sync_copy(ival_h, ival.at[pl.ds(0, N_EM)])
    pltpu.sync_copy(b_h, eb.at[pl.ds(0, N_EM)])
    pltpu.sync_copy(z_h, ez.at[pl.ds(0, N_EM)])
    pltpu.sync_copy(y_h, ey.at[pl.ds(0, N_EM)])
    pltpu.sync_copy(x_h, ex.at[pl.ds(0, N_EM)])
    pltpu.sync_copy(kyi_h, kyi)
    pltpu.sync_copy(kxi_h, kxi)
    pltpu.sync_copy(kyf_h, kyf)
    pltpu.sync_copy(kxf_h, kxf)
    pltpu.sync_copy(sig_h, sig)

    sig_v = sig[pl.ds(0, 16)]
    inv2s2 = 0.5 / (sig_v * sig_v)
    iota = lax.iota(jnp.int32, 16)
    iotaf = iota.astype(jnp.float32)
    zerov = jnp.zeros((16,), jnp.float32)

    for rep in range(REPS):
        slab_id = wid + rep * NWORK
        sb = slab_id // SLABS_PER_B
        z0 = (slab_id % SLABS_PER_B) * ZS
        pltpu.sync_copy(zero_h, slab)

        def scan(g, nh):
            bg = eb[pl.ds(g * 16, 16)]
            zg = ez[pl.ds(g * 16, 16)]
            m = (bg == sb) & (zg >= z0 - PAD) & (zg <= z0 + ZS - 1 + PAD)
            plsc.store_compressed(elist.at[pl.ds(nh, 16)], g * 16 + iota,
                                  mask=m)
            return nh + plsc.all_reduce_population_count(m)[0]

        nhits = lax.fori_loop(0, NGRP, scan, 0)

        def ebody(h, carry):
            e = _sload(elist, h)
            ze = _sload(ez, e)
            ye = _sload(ey, e)
            xe = _sload(ex, e)
            zos_e = _sload(zos, e)
            yos_e = _sload(yos, e)
            xos_e = _sload(xos, e)
            amp = SCALE_ * jnp.maximum(_sload(ival, e), 0.0)

            az[pl.ds(0, 16)] = zerov
            az[pl.ds(16, 16)] = zerov
            az[pl.ds(32, 16)] = zerov
            az[pl.ds(48, 16)] = zerov
            for c in range(2):
                k = iota + c * 16
                kf = iotaf + float(c * 16)
                dz = kf - (10.0 + zos_e)
                gzv = jnp.exp(-(dz * dz - zos_e * zos_e) * inv2s2) * amp
                az[pl.ds(AZOFF + c * 16, 16)] = jnp.where(k < PSF, gzv, 0.0)

            y0 = ye - PAD
            x0 = xe - PAD
            yos2xos2 = yos_e * yos_e + xos_e * xos_e
            for c in range(CHUNKS):
                yy = y0 + kyi[pl.ds(c * 16, 16)]
                xx = x0 + kxi[pl.ds(c * 16, 16)]
                dy = kyf[pl.ds(c * 16, 16)] - yos_e
                dx = kxf[pl.ds(c * 16, 16)] - xos_e
                v = jnp.exp(-(dy * dy + dx * dx - yos2xos2) * inv2s2)
                inb = (yy >= 0) & (yy < NH) & (xx >= 0) & (xx < ND)
                pval[pl.ds(c * 16, 16)] = jnp.where(inb, v, 0.0)
                pidx[pl.ds(c * 16, 16)] = jnp.where(inb, yy * ND + xx, 0)

            kz0 = z0 - ze + PAD + AZOFF
            for zloc in range(ZS):
                s = _sload(az, kz0 + zloc)
                tgt = slab.at[pl.ds(zloc * SLICE, SLICE)]
                for c in range(CHUNKS):
                    v = pval[pl.ds(c * 16, 16)] * s
                    plsc.addupdate_scatter(tgt, [pidx[pl.ds(c * 16, 16)]], v)
            return carry

        lax.fori_loop(0, nhits, ebody, 0)
        pltpu.sync_copy(slab, out_h.at[pl.ds(slab_id * SLAB, SLAB)])


def kernel(x_os_val, y_os_val, z_os_val, i_val, sigma, b, ch, z, y, x):
    del ch
    lin = jnp.arange(CHUNKS * 16, dtype=jnp.int32)
    kyi = jnp.where(lin < NPATCH, lin // PSF, 0)
    kxi = jnp.where(lin < NPATCH, lin % PSF, 1000)
    kyf = (kyi - PAD).astype(jnp.float32)
    kxf = (kxi - PAD).astype(jnp.float32)
    sig16 = jnp.full((16,), sigma, dtype=jnp.float32)
    zero = jnp.zeros((SLAB,), dtype=jnp.float32)

    mesh = plsc.VectorSubcoreMesh(core_axis_name="c", subcore_axis_name="s",
                                  num_cores=NC, num_subcores=NS)
    out = pl.kernel(
        _sc_body,
        out_type=jax.ShapeDtypeStruct((NB * NH * NW_ * ND,), jnp.float32),
        mesh=mesh,
        compiler_params=pltpu.CompilerParams(needs_layout_passes=False),
        scratch_types=[
            pltpu.VMEM((SLAB,), jnp.float32),
            pltpu.VMEM((EPAD,), jnp.float32),
            pltpu.VMEM((EPAD,), jnp.float32),
            pltpu.VMEM((EPAD,), jnp.float32),
            pltpu.VMEM((EPAD,), jnp.float32),
            pltpu.VMEM((EPAD,), jnp.int32),
            pltpu.VMEM((EPAD,), jnp.int32),
            pltpu.VMEM((EPAD,), jnp.int32),
            pltpu.VMEM((EPAD,), jnp.int32),
            pltpu.VMEM((CHUNKS * 16,), jnp.int32),
            pltpu.VMEM((CHUNKS * 16,), jnp.int32),
            pltpu.VMEM((CHUNKS * 16,), jnp.float32),
            pltpu.VMEM((CHUNKS * 16,), jnp.float32),
            pltpu.VMEM((AZLEN + 16,), jnp.float32),
            pltpu.VMEM((CHUNKS * 16,), jnp.float32),
            pltpu.VMEM((CHUNKS * 16,), jnp.int32),
            pltpu.VMEM((EPAD,), jnp.int32),
            pltpu.VMEM((16,), jnp.float32),
        ],
    )(x_os_val, y_os_val, z_os_val, i_val, sig16,
      b.astype(jnp.int32), z.astype(jnp.int32), y.astype(jnp.int32),
      x.astype(jnp.int32), kyi, kxi, kyf, kxf, zero)
    return out.reshape(NB, 1, NH, NW_, ND)


# speedup vs baseline: 225.9569x; 2.6693x over previous
import jax
import jax.numpy as jnp
from jax import lax
from jax.experimental import pallas as pl
from jax.experimental.pallas import tpu as pltpu
from jax.experimental.pallas import tpu_sc as plsc

N_EM = 2000
PSF = 21
PAD = PSF // 2
NB, NH, NW_, ND = 4, 128, 128, 64
SCALE_ = 10000.0
NC, NS = 2, 16
NWORK = NC * NS
ZS = 8
SLICE = NW_ * ND
SLAB = ZS * SLICE
NSLAB = (NB * NH) // ZS
SLABS_PER_B = NH // ZS
REPS = NSLAB // NWORK
NPATCH = PSF * PSF
CHUNKS = (NPATCH + 15) // 16
NGRP = N_EM // 16
EPAD = N_EM + 16
AZOFF = 16
AZLEN = 64


def _sload(ref, i):
    return ref[pl.ds(i, 16)][0]


def _sc_body(xos_h, yos_h, zos_h, ival_h, sig_h, b_h, z_h, y_h, x_h,
             kyi_h, kxi_h, kyf_h, kxf_h, zero_h, out_h,
             slab, xos, yos, zos, ival, eb, ez, ey, ex,
             kyi, kxi, kyf, kxf, az, pval, pidx, elist, sig):
    wid = lax.axis_index("s") * NC + lax.axis_index("c")

    pltpu.sync_copy(xos_h, xos.at[pl.ds(0, N_EM)])
    pltpu.sync_copy(yos_h, yos.at[pl.ds(0, N_EM)])
    pltpu.sync_copy(zos_h, zos.at[pl.ds(0, N_EM)])
    pltpu.sync_copy(ival_h, ival.at[pl.ds(0, N_EM)])
    pltpu.sync_copy(b_h, eb.at[pl.ds(0, N_EM)])
    pltpu.sync_copy(z_h, ez.at[pl.ds(0, N_EM)])
    pltpu.sync_copy(y_h, ey.at[pl.ds(0, N_EM)])
    pltpu.sync_copy(x_h, ex.at[pl.ds(0, N_EM)])
    pltpu.sync_copy(kyi_h, kyi)
    pltpu.sync_copy(kxi_h, kxi)
    pltpu.sync_copy(kyf_h, kyf)
    pltpu.sync_copy(kxf_h, kxf)
    pltpu.sync_copy(sig_h, sig)

    sig_v = sig[pl.ds(0, 16)]
    inv2s2 = 0.5 / (sig_v * sig_v)
    iota = lax.iota(jnp.int32, 16)
    iotaf = iota.astype(jnp.float32)
    zerov = jnp.zeros((16,), jnp.float32)

    for rep in range(REPS):
        slab_id = wid + rep * NWORK
        sb = slab_id // SLABS_PER_B
        z0 = (slab_id % SLABS_PER_B) * ZS
        pltpu.sync_copy(zero_h, slab)

        def scan(g, nh):
            bg = eb[pl.ds(g * 16, 16)]
            zg = ez[pl.ds(g * 16, 16)]
            m = (bg == sb) & (zg >= z0 - PAD) & (zg <= z0 + ZS - 1 + PAD)
            plsc.store_compressed(elist.at[pl.ds(nh, 16)], g * 16 + iota,
                                  mask=m)
            return nh + plsc.all_reduce_population_count(m)[0]

        nhits = lax.fori_loop(0, NGRP, scan, 0)

        def ebody(h, carry):
            e = _sload(elist, h)
            ze = _sload(ez, e)
            ye = _sload(ey, e)
            xe = _sload(ex, e)
            zos_e = _sload(zos, e)
            yos_e = _sload(yos, e)
            xos_e = _sload(xos, e)
            amp = SCALE_ * jnp.maximum(_sload(ival, e), 0.0)

            az[pl.ds(0, 16)] = zerov
            az[pl.ds(16, 16)] = zerov
            az[pl.ds(32, 16)] = zerov
            az[pl.ds(48, 16)] = zerov
            for c in range(2):
                k = iota + c * 16
                kf = iotaf + float(c * 16)
                dz = kf - (10.0 + zos_e)
                gzv = jnp.exp(-(dz * dz - zos_e * zos_e) * inv2s2) * amp
                az[pl.ds(AZOFF + c * 16, 16)] = jnp.where(k < PSF, gzv, 0.0)

            y0 = ye - PAD
            x0 = xe - PAD
            yos2xos2 = yos_e * yos_e + xos_e * xos_e
            for c in range(CHUNKS):
                yy = y0 + kyi[pl.ds(c * 16, 16)]
                xx = x0 + kxi[pl.ds(c * 16, 16)]
                dy = kyf[pl.ds(c * 16, 16)] - yos_e
                dx = kxf[pl.ds(c * 16, 16)] - xos_e
                v = jnp.exp(-(dy * dy + dx * dx - yos2xos2) * inv2s2)
                inb = (yy >= 0) & (yy < NH) & (xx >= 0) & (xx < ND)
                pval[pl.ds(c * 16, 16)] = jnp.where(inb, v, 0.0)
                pidx[pl.ds(c * 16, 16)] = jnp.where(inb, yy * ND + xx, iota)

            kz0 = z0 - ze + PAD + AZOFF
            svec = az[pl.ds(kz0, 16)]
            scales = [svec[zloc] for zloc in range(ZS)]
            for c in range(CHUNKS):
                v = pval[pl.ds(c * 16, 16)]
                ii = pidx[pl.ds(c * 16, 16)]
                for zloc in range(ZS):
                    tgt = slab.at[pl.ds(zloc * SLICE, SLICE)]
                    plsc.addupdate_scatter(tgt, [ii], v * scales[zloc])
            return carry

        lax.fori_loop(0, nhits, ebody, 0)
        pltpu.sync_copy(slab, out_h.at[pl.ds(slab_id * SLAB, SLAB)])


def kernel(x_os_val, y_os_val, z_os_val, i_val, sigma, b, ch, z, y, x):
    del ch
    lin = jnp.arange(CHUNKS * 16, dtype=jnp.int32)
    kyi = jnp.where(lin < NPATCH, lin // PSF, 0)
    kxi = jnp.where(lin < NPATCH, lin % PSF, 1000)
    kyf = (kyi - PAD).astype(jnp.float32)
    kxf = (kxi - PAD).astype(jnp.float32)
    sig16 = jnp.full((16,), sigma, dtype=jnp.float32)
    zero = jnp.zeros((SLAB,), dtype=jnp.float32)

    mesh = plsc.VectorSubcoreMesh(core_axis_name="c", subcore_axis_name="s",
                                  num_cores=NC, num_subcores=NS)
    out = pl.kernel(
        _sc_body,
        out_type=jax.ShapeDtypeStruct((NB * NH * NW_ * ND,), jnp.float32),
        mesh=mesh,
        compiler_params=pltpu.CompilerParams(needs_layout_passes=False),
        scratch_types=[
            pltpu.VMEM((SLAB,), jnp.float32),
            pltpu.VMEM((EPAD,), jnp.float32),
            pltpu.VMEM((EPAD,), jnp.float32),
            pltpu.VMEM((EPAD,), jnp.float32),
            pltpu.VMEM((EPAD,), jnp.float32),
            pltpu.VMEM((EPAD,), jnp.int32),
            pltpu.VMEM((EPAD,), jnp.int32),
            pltpu.VMEM((EPAD,), jnp.int32),
            pltpu.VMEM((EPAD,), jnp.int32),
            pltpu.VMEM((CHUNKS * 16,), jnp.int32),
            pltpu.VMEM((CHUNKS * 16,), jnp.int32),
            pltpu.VMEM((CHUNKS * 16,), jnp.float32),
            pltpu.VMEM((CHUNKS * 16,), jnp.float32),
            pltpu.VMEM((AZLEN + 16,), jnp.float32),
            pltpu.VMEM((CHUNKS * 16,), jnp.float32),
            pltpu.VMEM((CHUNKS * 16,), jnp.int32),
            pltpu.VMEM((EPAD,), jnp.int32),
            pltpu.VMEM((16,), jnp.float32),
        ],
    )(x_os_val, y_os_val, z_os_val, i_val, sig16,
      b.astype(jnp.int32), z.astype(jnp.int32), y.astype(jnp.int32),
      x.astype(jnp.int32), kyi, kxi, kyf, kxf, zero)
    return out.reshape(NB, 1, NH, NW_, ND)


# speedup vs baseline: 234.9577x; 1.0398x over previous
import jax
import jax.numpy as jnp
from jax import lax
from jax.experimental import pallas as pl
from jax.experimental.pallas import tpu as pltpu
from jax.experimental.pallas import tpu_sc as plsc

N_EM = 2000
PSF = 21
PAD = PSF // 2
NB, NH, NW_, ND = 4, 128, 128, 64
SCALE_ = 10000.0
NC, NS = 2, 16
NWORK = NC * NS
ZS = 8
SLICE = NW_ * ND
SLAB = ZS * SLICE
NSLAB = (NB * NH) // ZS
SLABS_PER_B = NH // ZS
REPS = NSLAB // NWORK
NPATCH = PSF * PSF
CHUNKS = (NPATCH + 15) // 16
NGRP = N_EM // 16
EPAD = N_EM + 16
AZOFF = 16
AZLEN = 64


def _sload(ref, i):
    return ref[pl.ds(i, 16)][0]


def _sc_body(xos_h, yos_h, zos_h, ival_h, sig_h, b_h, z_h, y_h, x_h,
             kyi_h, kxi_h, kyf_h, kxf_h, k2t_h, zero_h, out_h,
             slab, xos, yos, zos, ival, eb, ez, ey, ex,
             kyi, kxi, kyf, kxf, k2t, elist, sig):
    wid = lax.axis_index("s") * NC + lax.axis_index("c")

    pltpu.sync_copy(xos_h, xos.at[pl.ds(0, N_EM)])
    pltpu.sync_copy(yos_h, yos.at[pl.ds(0, N_EM)])
    pltpu.sync_copy(zos_h, zos.at[pl.ds(0, N_EM)])
    pltpu.sync_copy(ival_h, ival.at[pl.ds(0, N_EM)])
    pltpu.sync_copy(b_h, eb.at[pl.ds(0, N_EM)])
    pltpu.sync_copy(z_h, ez.at[pl.ds(0, N_EM)])
    pltpu.sync_copy(y_h, ey.at[pl.ds(0, N_EM)])
    pltpu.sync_copy(x_h, ex.at[pl.ds(0, N_EM)])
    pltpu.sync_copy(kyi_h, kyi)
    pltpu.sync_copy(kxi_h, kxi)
    pltpu.sync_copy(kyf_h, kyf)
    pltpu.sync_copy(kxf_h, kxf)
    pltpu.sync_copy(k2t_h, k2t)
    pltpu.sync_copy(sig_h, sig)

    sig_v = sig[pl.ds(0, 16)]
    inv2s2 = 0.5 / (sig_v * sig_v)
    inv2s2_s = inv2s2[0]
    iota = lax.iota(jnp.int32, 16)
    iotaf = iota.astype(jnp.float32)
    zerov = jnp.zeros((16,), jnp.float32)

    for rep in range(REPS):
        slab_id = wid + rep * NWORK
        sb = slab_id // SLABS_PER_B
        z0 = (slab_id % SLABS_PER_B) * ZS
        pltpu.sync_copy(zero_h, slab)

        def scan(g, nh):
            bg = eb[pl.ds(g * 16, 16)]
            zg = ez[pl.ds(g * 16, 16)]
            m = (bg == sb) & (zg >= z0 - PAD) & (zg <= z0 + ZS - 1 + PAD)
            plsc.store_compressed(elist.at[pl.ds(nh, 16)], g * 16 + iota,
                                  mask=m)
            return nh + plsc.all_reduce_population_count(m)[0]

        nhits = lax.fori_loop(0, NGRP, scan, 0)

        def ebody(h, carry):
            e = _sload(elist, h)
            ze = _sload(ez, e)
            ye = _sload(ey, e)
            xe = _sload(ex, e)
            zos_e = _sload(zos, e)
            yos_e = _sload(yos, e)
            xos_e = _sload(xos, e)
            amp = SCALE_ * jnp.maximum(_sload(ival, e), 0.0)

            t = (z0 - ze) + iota

            tf = t.astype(jnp.float32)
            dz = tf - zos_e
            gzv = jnp.exp(-(dz * dz - zos_e * zos_e) * inv2s2) * amp
            svec = jnp.where((t >= -PAD) & (t <= PAD), gzv, 0.0)
            scales = [svec[zloc] for zloc in range(ZS)]

            y0 = ye - PAD
            x0 = xe - PAD
            c1s = 2.0 * yos_e * inv2s2_s
            c2s = 2.0 * xos_e * inv2s2_s
            c3s = -inv2s2_s

            def build(c):
                yy = y0 + kyi[pl.ds(c * 16, 16)]
                xx = x0 + kxi[pl.ds(c * 16, 16)]
                e2 = (c3s * k2t[pl.ds(c * 16, 16)]
                      + c1s * kyf[pl.ds(c * 16, 16)]
                      + c2s * kxf[pl.ds(c * 16, 16)])
                v = jnp.exp(e2)
                inb = (yy >= 0) & (yy < NH) & (xx >= 0) & (xx < ND)
                v = jnp.where(inb, v, 0.0)
                ii = jnp.where(inb, yy * ND + xx, iota)
                return v, ii

            def scatter(v, ii):
                for zloc in range(ZS):
                    tgt = slab.at[pl.ds(zloc * SLICE, SLICE)]
                    plsc.addupdate_scatter(tgt, [ii], v * scales[zloc])

            for base in range(0, CHUNKS, 2):
                va, ia = build(base)
                vb, ib = build(base + 1)
                scatter(va, ia)
                scatter(vb, ib)
            return carry

        lax.fori_loop(0, nhits, ebody, 0)
        pltpu.sync_copy(slab, out_h.at[pl.ds(slab_id * SLAB, SLAB)])


def kernel(x_os_val, y_os_val, z_os_val, i_val, sigma, b, ch, z, y, x):
    del ch
    lin = jnp.arange(CHUNKS * 16, dtype=jnp.int32)
    kyi = jnp.where(lin < NPATCH, lin // PSF, 0)
    kxi = jnp.where(lin < NPATCH, lin % PSF, 1000)
    kyf = (kyi - PAD).astype(jnp.float32)
    kxf = (kxi - PAD).astype(jnp.float32)
    k2t = kyf * kyf + kxf * kxf
    sig16 = jnp.full((16,), sigma, dtype=jnp.float32)
    zero = jnp.zeros((SLAB,), dtype=jnp.float32)

    mesh = plsc.VectorSubcoreMesh(core_axis_name="c", subcore_axis_name="s",
                                  num_cores=NC, num_subcores=NS)
    out = pl.kernel(
        _sc_body,
        out_type=jax.ShapeDtypeStruct((NB * NH * NW_ * ND,), jnp.float32),
        mesh=mesh,
        compiler_params=pltpu.CompilerParams(needs_layout_passes=False),
        scratch_types=[
            pltpu.VMEM((SLAB,), jnp.float32),
            pltpu.VMEM((EPAD,), jnp.float32),
            pltpu.VMEM((EPAD,), jnp.float32),
            pltpu.VMEM((EPAD,), jnp.float32),
            pltpu.VMEM((EPAD,), jnp.float32),
            pltpu.VMEM((EPAD,), jnp.int32),
            pltpu.VMEM((EPAD,), jnp.int32),
            pltpu.VMEM((EPAD,), jnp.int32),
            pltpu.VMEM((EPAD,), jnp.int32),
            pltpu.VMEM((CHUNKS * 16,), jnp.int32),
            pltpu.VMEM((CHUNKS * 16,), jnp.int32),
            pltpu.VMEM((CHUNKS * 16,), jnp.float32),
            pltpu.VMEM((CHUNKS * 16,), jnp.float32),
            pltpu.VMEM((CHUNKS * 16,), jnp.float32),
            pltpu.VMEM((EPAD,), jnp.int32),
            pltpu.VMEM((16,), jnp.float32),
        ],
    )(x_os_val, y_os_val, z_os_val, i_val, sig16,
      b.astype(jnp.int32), z.astype(jnp.int32), y.astype(jnp.int32),
      x.astype(jnp.int32), kyi, kxi, kyf, kxf, k2t, zero)
    return out.reshape(NB, 1, NH, NW_, ND)


# speedup vs baseline: 303.7499x; 1.2928x over previous
import jax
import jax.numpy as jnp
from jax import lax
from jax.experimental import pallas as pl
from jax.experimental.pallas import tpu as pltpu
from jax.experimental.pallas import tpu_sc as plsc

N_EM = 2000
PSF = 21
PAD = PSF // 2
NB, NH, NW_, ND = 4, 128, 128, 64
SCALE_ = 10000.0
NC, NS = 2, 16
NWORK = NC * NS
ZS = 8
SLICE = NW_ * ND
SLAB = ZS * SLICE
NSLAB = (NB * NH) // ZS
SLABS_PER_B = NH // ZS
REPS = NSLAB // NWORK
NPATCH = PSF * PSF
CHUNKS = (NPATCH + 15) // 16
NGRP = N_EM // 16
EPAD = N_EM + 16
AZOFF = 16
AZLEN = 64


def _sload(ref, i):
    return ref[pl.ds(i, 16)][0]


def _sc_body(xos_h, yos_h, zos_h, ival_h, sig_h, b_h, z_h, y_h, x_h,
             kyi_h, kxi_h, kyf_h, kxf_h, k2t_h, zero_h, out_h,
             slab, xos, yos, zos, ival, eb, ez, ey, ex,
             kyi, kxi, kyf, kxf, k2t, elist, sig):
    wid = lax.axis_index("s") * NC + lax.axis_index("c")

    pltpu.sync_copy(xos_h, xos.at[pl.ds(0, N_EM)])
    pltpu.sync_copy(yos_h, yos.at[pl.ds(0, N_EM)])
    pltpu.sync_copy(zos_h, zos.at[pl.ds(0, N_EM)])
    pltpu.sync_copy(ival_h, ival.at[pl.ds(0, N_EM)])
    pltpu.sync_copy(b_h, eb.at[pl.ds(0, N_EM)])
    pltpu.sync_copy(z_h, ez.at[pl.ds(0, N_EM)])
    pltpu.sync_copy(y_h, ey.at[pl.ds(0, N_EM)])
    pltpu.sync_copy(x_h, ex.at[pl.ds(0, N_EM)])
    pltpu.sync_copy(kyi_h, kyi)
    pltpu.sync_copy(kxi_h, kxi)
    pltpu.sync_copy(kyf_h, kyf)
    pltpu.sync_copy(kxf_h, kxf)
    pltpu.sync_copy(k2t_h, k2t)
    pltpu.sync_copy(sig_h, sig)

    sig_v = sig[pl.ds(0, 16)]
    inv2s2 = 0.5 / (sig_v * sig_v)
    inv2s2_s = inv2s2[0]
    iota = lax.iota(jnp.int32, 16)
    iotaf = iota.astype(jnp.float32)
    zerov = jnp.zeros((16,), jnp.float32)

    for rep in range(REPS):
        slab_id = wid + rep * NWORK
        sb = slab_id // SLABS_PER_B
        z0 = (slab_id % SLABS_PER_B) * ZS
        pltpu.sync_copy(zero_h, slab)

        def scan(g, nh):
            bg = eb[pl.ds(g * 16, 16)]
            zg = ez[pl.ds(g * 16, 16)]
            m = (bg == sb) & (zg >= z0 - PAD) & (zg <= z0 + ZS - 1 + PAD)
            plsc.store_compressed(elist.at[pl.ds(nh, 16)], g * 16 + iota,
                                  mask=m)
            return nh + plsc.all_reduce_population_count(m)[0]

        nhits = lax.fori_loop(0, NGRP, scan, 0)

        @plsc.parallel_loop(0, nhits, 1, unroll=2)
        def ebody(h):
            e = _sload(elist, h)
            ze = _sload(ez, e)
            ye = _sload(ey, e)
            xe = _sload(ex, e)
            zos_e = _sload(zos, e)
            yos_e = _sload(yos, e)
            xos_e = _sload(xos, e)
            amp = SCALE_ * jnp.maximum(_sload(ival, e), 0.0)

            t = (z0 - ze) + iota

            tf = t.astype(jnp.float32)
            dz = tf - zos_e
            gzv = jnp.exp(-(dz * dz - zos_e * zos_e) * inv2s2) * amp
            svec = jnp.where((t >= -PAD) & (t <= PAD), gzv, 0.0)
            scales = [svec[zloc] for zloc in range(ZS)]

            c1s = 2.0 * yos_e * inv2s2_s
            c2s = 2.0 * xos_e * inv2s2_s
            c3s = -inv2s2_s

            def build(c):
                fy = kyf[pl.ds(c * 16, 16)]
                fx = kxf[pl.ds(c * 16, 16)]
                yy = ye + fy.astype(jnp.int32)
                xx = xe + fx.astype(jnp.int32)
                e2 = fy * (c3s * fy + c1s) + fx * (c3s * fx + c2s)
                v = jnp.exp(e2)
                inb = (yy >= 0) & (yy < NH) & (xx >= 0) & (xx < ND)
                v = jnp.where(inb, v, 0.0)
                ii = jnp.where(inb, yy * ND + xx, iota)
                return v, ii

            @plsc.parallel_loop(0, CHUNKS, 1, unroll=4)
            def chunk_loop(c):
                v, ii = build(c)
                for zloc in range(ZS):
                    tgt = slab.at[pl.ds(zloc * SLICE, SLICE)]
                    plsc.addupdate_scatter(tgt, [ii], v * scales[zloc])

        pltpu.sync_copy(slab, out_h.at[pl.ds(slab_id * SLAB, SLAB)])


def kernel(x_os_val, y_os_val, z_os_val, i_val, sigma, b, ch, z, y, x):
    del ch
    lin = jnp.arange(CHUNKS * 16, dtype=jnp.int32)
    kyi = jnp.where(lin < NPATCH, lin // PSF, 0)
    kxi = jnp.where(lin < NPATCH, lin % PSF, 1000)
    kyf = (kyi - PAD).astype(jnp.float32)
    kxf = (kxi - PAD).astype(jnp.float32)
    k2t = kyf * kyf + kxf * kxf
    sig16 = jnp.full((16,), sigma, dtype=jnp.float32)
    zero = jnp.zeros((SLAB,), dtype=jnp.float32)

    mesh = plsc.VectorSubcoreMesh(core_axis_name="c", subcore_axis_name="s",
                                  num_cores=NC, num_subcores=NS)
    out = pl.kernel(
        _sc_body,
        out_type=jax.ShapeDtypeStruct((NB * NH * NW_ * ND,), jnp.float32),
        mesh=mesh,
        compiler_params=pltpu.CompilerParams(needs_layout_passes=False),
        scratch_types=[
            pltpu.VMEM((SLAB,), jnp.float32),
            pltpu.VMEM((EPAD,), jnp.float32),
            pltpu.VMEM((EPAD,), jnp.float32),
            pltpu.VMEM((EPAD,), jnp.float32),
            pltpu.VMEM((EPAD,), jnp.float32),
            pltpu.VMEM((EPAD,), jnp.int32),
            pltpu.VMEM((EPAD,), jnp.int32),
            pltpu.VMEM((EPAD,), jnp.int32),
            pltpu.VMEM((EPAD,), jnp.int32),
            pltpu.VMEM((CHUNKS * 16,), jnp.int32),
            pltpu.VMEM((CHUNKS * 16,), jnp.int32),
            pltpu.VMEM((CHUNKS * 16,), jnp.float32),
            pltpu.VMEM((CHUNKS * 16,), jnp.float32),
            pltpu.VMEM((CHUNKS * 16,), jnp.float32),
            pltpu.VMEM((EPAD,), jnp.int32),
            pltpu.VMEM((16,), jnp.float32),
        ],
    )(x_os_val, y_os_val, z_os_val, i_val, sig16,
      b.astype(jnp.int32), z.astype(jnp.int32), y.astype(jnp.int32),
      x.astype(jnp.int32), kyi, kxi, kyf, kxf, k2t, zero)
    return out.reshape(NB, 1, NH, NW_, ND)
